# baseline (device time: 46480 ns/iter reference)
import jax
import jax.numpy as jnp
from jax import lax
from jax.experimental import pallas as pl
from jax.experimental.pallas import tpu as pltpu

N_DEV = 4
N_TOK = 2048
D_IN = 512
D_OUT = 1024
N_EXP = 16
EXP_PER_DEV = N_EXP // N_DEV
ROWS = N_TOK // N_DEV
K_ALL = EXP_PER_DEV * D_IN
SUB = 4
HROWS = ROWS // SUB
N_SLOTS = (N_DEV - 1) * SUB


def kernel(x, router_W, route_idx, expert_W):
    def body(x_ref, rw_ref, idx_ref, ew_ref, out_ref,
             xg_ref, w_ref, send_ref, recv_ref, send_sems, recv_sems):
        my = lax.axis_index("i")

        barrier_sem = pltpu.get_barrier_semaphore()
        for o in range(1, N_DEV):
            peer = lax.rem(my + o, N_DEV)
            pl.semaphore_signal(
                barrier_sem, inc=1,
                device_id=(peer,), device_id_type=pltpu.DeviceIdType.MESH,
            )

        rwb = rw_ref[:, :].astype(jnp.bfloat16)

        for le in range(EXP_PER_DEV):
            w_ref[le * D_IN:(le + 1) * D_IN, :] = ew_ref[le, :, :].astype(
                jnp.bfloat16)
        wv = w_ref[:, :]

        def prep_chunk(r):
            xbc = x_ref[pl.ds(r, ROWS), :].astype(jnp.bfloat16)
            scores = jnp.dot(xbc, rwb, preferred_element_type=jnp.float32)
            m = jnp.max(scores, axis=1, keepdims=True)
            p = jnp.exp(scores - m)
            p = p / jnp.sum(p, axis=1, keepdims=True)
            i0 = idx_ref[pl.ds(r, ROWS), 0:1]
            i1 = idx_ref[pl.ds(r, ROWS), 1:2]
            iota = lax.broadcasted_iota(jnp.int32, (ROWS, N_EXP), 1)
            g0 = jnp.sum(jnp.where(iota == i0, p, 0.0), axis=1,
                         keepdims=True)
            g1 = jnp.sum(jnp.where(iota == i1, p, 0.0), axis=1,
                         keepdims=True)
            gs = g0 + g1
            w0 = g0 / gs
            w1 = g1 / gs
            for le in range(EXP_PER_DEV):
                gid = my * EXP_PER_DEV + le
                g = (jnp.where(i0 == gid, w0, 0.0)
                     + jnp.where(i1 == gid, w1, 0.0)).astype(jnp.bfloat16)
                xg_ref[pl.ds(r, ROWS), le * D_IN:(le + 1) * D_IN] = xbc * g

        rdmas = []
        for o in range(1, N_DEV):
            dst = lax.rem(my + o, N_DEV)
            r = dst * ROWS
            prep_chunk(r)
            for h in range(SUB):
                slot = (o - 1) * SUB + h
                send_ref[slot, :, :] = jnp.dot(
                    xg_ref[pl.ds(r + h * HROWS, HROWS), :], wv,
                    preferred_element_type=jnp.float32,
                ).astype(jnp.bfloat16)
                if o == 1 and h == 0:
                    pl.semaphore_wait(barrier_sem, N_DEV - 1)
                rdma = pltpu.make_async_remote_copy(
                    src_ref=send_ref.at[slot],
                    dst_ref=recv_ref.at[slot],
                    send_sem=send_sems.at[slot],
                    recv_sem=recv_sems.at[slot],
                    device_id=(dst,),
                    device_id_type=pltpu.DeviceIdType.MESH,
                )
                rdma.start()
                rdmas.append(rdma)

        r = my * ROWS
        prep_chunk(r)
        out_ref[:, :] = jnp.dot(xg_ref[pl.ds(r, ROWS), :], wv,
                                preferred_element_type=jnp.float32)
        for o in range(1, N_DEV):
            for h in range(SUB):
                slot = (o - 1) * SUB + h
                rdmas[slot].wait_recv()
                out_ref[h * HROWS:(h + 1) * HROWS, :] += recv_ref[
                    slot, :, :].astype(jnp.float32)

        for rd in rdmas:
            rd.wait_send()

    return pl.pallas_call(
        body,
        out_shape=jax.ShapeDtypeStruct((ROWS, D_OUT), jnp.float32),
        in_specs=[
            pl.BlockSpec(memory_space=pltpu.VMEM),
            pl.BlockSpec(memory_space=pltpu.VMEM),
            pl.BlockSpec(memory_space=pltpu.VMEM),
            pl.BlockSpec(memory_space=pltpu.VMEM),
        ],
        out_specs=pl.BlockSpec(memory_space=pltpu.VMEM),
        scratch_shapes=[
            pltpu.VMEM((N_TOK, K_ALL), jnp.bfloat16),
            pltpu.VMEM((K_ALL, D_OUT), jnp.bfloat16),
            pltpu.VMEM((N_SLOTS, HROWS, D_OUT), jnp.bfloat16),
            pltpu.VMEM((N_SLOTS, HROWS, D_OUT), jnp.bfloat16),
            pltpu.SemaphoreType.DMA((N_SLOTS,)),
            pltpu.SemaphoreType.DMA((N_SLOTS,)),
        ],
        compiler_params=pltpu.CompilerParams(
            collective_id=0,
            vmem_limit_bytes=128 * 1024 * 1024,
        ),
    )(x, router_W, route_idx, expert_W)
